# Initial kernel scaffold; baseline (speedup 1.0000x reference)
import jax
import jax.numpy as jnp
from jax.experimental import pallas as pl


def kernel(relative_position_bias_table):
    raise NotImplementedError("write your pallas kernel here")


# speedup vs baseline: 1.0330x; 1.0330x over previous
import functools

import numpy as np
import jax
import jax.numpy as jnp
from jax import lax
from jax.experimental import pallas as pl
from jax.experimental.pallas import tpu as pltpu
from jax.experimental.pallas import tpu_sc as plsc

_WIN = 12
_SEQ = _WIN * _WIN
_P = _SEQ * _SEQ
_NH = 32
_TBL = (2 * _WIN - 1) ** 2
_NVEC = _P // 16
_UNROLL = 16


def _static_index() -> np.ndarray:
    coords = np.stack(np.meshgrid(np.arange(_WIN), np.arange(_WIN), indexing="ij"))
    flat = coords.reshape(2, -1)
    rel = (flat[:, :, None] - flat[:, None, :]).transpose(1, 2, 0)
    rel[:, :, 0] += _WIN - 1
    rel[:, :, 1] += _WIN - 1
    rel[:, :, 0] *= 2 * _WIN - 1
    return rel.sum(-1).reshape(-1).astype(np.int32)


_IDX32 = _static_index() * _NH


@functools.cache
def _build_sc_kernel():
    mesh = plsc.VectorSubcoreMesh(core_axis_name="c", subcore_axis_name="s")

    @functools.partial(
        pl.kernel,
        mesh=mesh,
        out_type=jax.ShapeDtypeStruct((_NH * _P,), jnp.float32),
        compiler_params=pltpu.CompilerParams(needs_layout_passes=False),
        scratch_types=[
            pltpu.VMEM((_TBL * _NH,), jnp.float32),
            pltpu.VMEM((_P,), jnp.int32),
            pltpu.VMEM((_P,), jnp.float32),
        ],
    )
    def _sc_bias_gather(table_hbm, idx_hbm, out_hbm, tbl_v, idx_v, row_v):
        w = lax.axis_index("s") * mesh.num_cores + lax.axis_index("c")
        pltpu.sync_copy(table_hbm, tbl_v)
        pltpu.sync_copy(idx_hbm, idx_v)

        def body(i, carry):
            for u in range(_UNROLL):
                off = (i * _UNROLL + u) * 16
                base = idx_v[pl.ds(off, 16)]
                row_v[pl.ds(off, 16)] = plsc.load_gather(tbl_v, [base + w])
            return carry

        lax.fori_loop(0, _NVEC // _UNROLL, body, 0)
        pltpu.sync_copy(row_v, out_hbm.at[pl.ds(w * _P, _P)])

    return _sc_bias_gather


def kernel(relative_position_bias_table):
    table_flat = relative_position_bias_table.reshape(-1)
    out = _build_sc_kernel()(table_flat, jnp.asarray(_IDX32))
    return out.reshape(_NH, _SEQ, _SEQ)


# speedup vs baseline: 1.1038x; 1.0686x over previous
import functools

import numpy as np
import jax
import jax.numpy as jnp
from jax import lax
from jax.experimental import pallas as pl
from jax.experimental.pallas import tpu as pltpu
from jax.experimental.pallas import tpu_sc as plsc

_WIN = 12
_SEQ = _WIN * _WIN
_P = _SEQ * _SEQ
_NH = 32
_TBL = (2 * _WIN - 1) ** 2
_NVEC = _P // 16
_UNROLL = 16


def _static_index() -> np.ndarray:
    coords = np.stack(np.meshgrid(np.arange(_WIN), np.arange(_WIN), indexing="ij"))
    flat = coords.reshape(2, -1)
    rel = (flat[:, :, None] - flat[:, None, :]).transpose(1, 2, 0)
    rel[:, :, 0] += _WIN - 1
    rel[:, :, 1] += _WIN - 1
    rel[:, :, 0] *= 2 * _WIN - 1
    return rel.sum(-1).reshape(-1).astype(np.int32)


_IDX32 = _static_index() * _NH


@functools.cache
def _build_sc_kernel():
    mesh = plsc.VectorSubcoreMesh(core_axis_name="c", subcore_axis_name="s")

    @functools.partial(
        pl.kernel,
        mesh=mesh,
        out_type=jax.ShapeDtypeStruct((_NH * _P,), jnp.float32),
        compiler_params=pltpu.CompilerParams(needs_layout_passes=False),
        scratch_types=[
            pltpu.VMEM((_TBL * _NH,), jnp.float32),
            pltpu.VMEM((_P,), jnp.int32),
            pltpu.VMEM((_P,), jnp.float32),
        ],
    )
    def _sc_bias_gather(table_hbm, idx_hbm, out_hbm, tbl_v, idx_v, row_v):
        w = lax.axis_index("s") * mesh.num_cores + lax.axis_index("c")
        pltpu.sync_copy(table_hbm, tbl_v)
        pltpu.sync_copy(idx_hbm, idx_v)

        @plsc.parallel_loop(0, _P, step=16, unroll=_UNROLL)
        def _gather_body(off):
            base = idx_v[pl.ds(off, 16)]
            row_v[pl.ds(off, 16)] = plsc.load_gather(tbl_v, [base + w])
        pltpu.sync_copy(row_v, out_hbm.at[pl.ds(w * _P, _P)])

    return _sc_bias_gather


def kernel(relative_position_bias_table):
    table_flat = relative_position_bias_table.reshape(-1)
    out = _build_sc_kernel()(table_flat, jnp.asarray(_IDX32))
    return out.reshape(_NH, _SEQ, _SEQ)
